# Initial kernel scaffold; baseline (speedup 1.0000x reference)
import jax
import jax.numpy as jnp
from jax.experimental import pallas as pl


def kernel(x, token_type_ids, pos_ids, tile_table, type_table, pos_table, gamma, beta):
    raise NotImplementedError("write your pallas kernel here")


# speedup vs baseline: 4.0492x; 4.0492x over previous
import functools

import jax
import jax.numpy as jnp
from jax import lax
from jax.experimental import pallas as pl
from jax.experimental.pallas import tpu as pltpu
from jax.experimental.pallas import tpu_sc as plsc

B, L, D = 4096, 200, 128
V, T, P = 1000, 31, 512
N = B * L
EPS = 1e-12

NC, NS = 2, 16
NW = NC * NS
C = 128
PER_W = N // NW
NCHUNK = PER_W // C


def _rsqrt(x):
    i = lax.bitcast_convert_type(x, jnp.int32)
    i = jnp.int32(0x5F3759DF) - (i >> 1)
    y = lax.bitcast_convert_type(i, jnp.float32)
    for _ in range(3):
        y = y * (1.5 - 0.5 * x * y * y)
    return y


_GATHER_DNUMS = lax.GatherDimensionNumbers(
    offset_dims=(), collapsed_slice_dims=(0,), start_index_map=(0,))


def _shuffle(v, idx):
    return lax.gather(v, idx[:, None], dimension_numbers=_GATHER_DNUMS,
                      slice_sizes=(1,),
                      mode=lax.GatherScatterMode.PROMISE_IN_BOUNDS)


def _hsum(v):
    for sh in (8, 4, 2, 1):
        idx = (lax.iota(jnp.int32, 16) + sh) & 15
        v = v + _shuffle(v, idx)
    return v


def _sc_body(x_hbm, tt_hbm, pp_hbm, tileT, typeT, posT, gamma_hbm, beta_hbm,
             out_hbm, xi, ti, pi, bufA, bufB, bufC, obuf, gv, bv,
             semA, semB, semC):
    wid = lax.axis_index("s") * NC + lax.axis_index("c")

    pltpu.sync_copy(gamma_hbm, gv)
    pltpu.sync_copy(beta_hbm, bv)

    def chunk_body(g, _):
        base = wid * PER_W + g * C
        pltpu.sync_copy(x_hbm.at[pl.ds(base, C)], xi)
        pltpu.sync_copy(tt_hbm.at[pl.ds(base, C)], ti)
        pltpu.sync_copy(pp_hbm.at[pl.ds(base, C)], pi)
        cpA = pltpu.async_copy(tileT.at[xi], bufA, semA)
        cpB = pltpu.async_copy(typeT.at[ti], bufB, semB)
        cpC = pltpu.async_copy(posT.at[pi], bufC, semC)
        cpA.wait()
        cpB.wait()
        cpC.wait()

        def token_body(t, _):
            s = jnp.zeros((16,), jnp.float32)
            q = jnp.zeros((16,), jnp.float32)
            for d8 in range(8):
                sl = pl.ds(d8 * 16, 16)
                v = bufA[t, sl] + bufB[t, sl] + bufC[t, sl]
                obuf[t, sl] = v
                s = s + v
                q = q + v * v
            mean = _hsum(s) * (1.0 / 128.0)
            var = _hsum(q) * (1.0 / 128.0) - mean * mean
            rstd = _rsqrt(var + EPS)
            for d8 in range(8):
                sl = pl.ds(d8 * 16, 16)
                v = obuf[t, sl]
                obuf[t, sl] = (v - mean) * rstd * gv[sl] + bv[sl]
            return 0

        lax.fori_loop(0, C, token_body, 0)
        pltpu.sync_copy(obuf, out_hbm.at[pl.ds(base, C)])
        return 0

    lax.fori_loop(0, NCHUNK, chunk_body, 0)


@jax.jit
def _run(x, tt, pp, tileT, typeT, posT, gamma, beta):
    mesh = plsc.VectorSubcoreMesh(core_axis_name="c", subcore_axis_name="s")
    kfn = pl.kernel(
        _sc_body,
        mesh=mesh,
        out_type=jax.ShapeDtypeStruct((N, D), jnp.float32),
        scratch_types=[
            pltpu.VMEM((C,), jnp.int32),
            pltpu.VMEM((C,), jnp.int32),
            pltpu.VMEM((C,), jnp.int32),
            pltpu.VMEM((C, D), jnp.float32),
            pltpu.VMEM((C, D), jnp.float32),
            pltpu.VMEM((C, D), jnp.float32),
            pltpu.VMEM((C, D), jnp.float32),
            pltpu.VMEM((D,), jnp.float32),
            pltpu.VMEM((D,), jnp.float32),
            pltpu.SemaphoreType.DMA,
            pltpu.SemaphoreType.DMA,
            pltpu.SemaphoreType.DMA,
        ],
    )
    return kfn(x, tt, pp, tileT, typeT, posT, gamma, beta)


def kernel(x, token_type_ids, pos_ids, tile_table, type_table, pos_table,
           gamma, beta):
    xf = x.reshape(N).astype(jnp.int32)
    tf = token_type_ids.reshape(N).astype(jnp.int32)
    pf = pos_ids.reshape(N).astype(jnp.int32)
    out = _run(xf, tf, pf, tile_table, type_table, pos_table, gamma, beta)
    return out.reshape(B, L, D)


# speedup vs baseline: 4.3135x; 1.0652x over previous
import functools

import jax
import jax.numpy as jnp
from jax import lax
from jax.experimental import pallas as pl
from jax.experimental.pallas import tpu as pltpu
from jax.experimental.pallas import tpu_sc as plsc

B, L, D = 4096, 200, 128
V, T, P = 1000, 31, 512
N = B * L
EPS = 1e-12

NC, NS = 2, 16
NW = NC * NS
C = 128
PER_W = N // NW
NCHUNK = PER_W // C


def _rsqrt(x):
    i = lax.bitcast_convert_type(x, jnp.int32)
    i = jnp.int32(0x5F3759DF) - (i >> 1)
    y = lax.bitcast_convert_type(i, jnp.float32)
    for _ in range(3):
        y = y * (1.5 - 0.5 * x * y * y)
    return y


def _tree_sum(xs):
    xs = list(xs)
    while len(xs) > 1:
        nxt = [xs[i] + xs[i + 1] for i in range(0, len(xs) - 1, 2)]
        if len(xs) % 2:
            nxt.append(xs[-1])
        xs = nxt
    return xs[0]


_GATHER_DNUMS = lax.GatherDimensionNumbers(
    offset_dims=(), collapsed_slice_dims=(0,), start_index_map=(0,))


def _shuffle(v, idx):
    return lax.gather(v, idx[:, None], dimension_numbers=_GATHER_DNUMS,
                      slice_sizes=(1,),
                      mode=lax.GatherScatterMode.PROMISE_IN_BOUNDS)


def _hsum(v):
    for sh in (8, 4, 2, 1):
        idx = (lax.iota(jnp.int32, 16) + sh) & 15
        v = v + _shuffle(v, idx)
    return v


def _sc_body(x_hbm, tt_hbm, pp_hbm, tileT, typeT, posT, gamma_hbm, beta_hbm,
             out_hbm, xi, ti, pi, bufA, bufB, bufC, gv, bv,
             semA0, semA1, semB0, semB1, semC0, semC1, semO0, semO1):
    wid = lax.axis_index("s") * NC + lax.axis_index("c")
    w0 = wid * PER_W
    semA = (semA0, semA1)
    semB = (semB0, semB1)
    semC = (semC0, semC1)
    semO = (semO0, semO1)

    pltpu.sync_copy(gamma_hbm, gv)
    pltpu.sync_copy(beta_hbm, bv)

    def fire(gg, b):
        base = w0 + gg * C
        pltpu.sync_copy(x_hbm.at[pl.ds(base, C)], xi.at[b])
        pltpu.sync_copy(tt_hbm.at[pl.ds(base, C)], ti.at[b])
        pltpu.sync_copy(pp_hbm.at[pl.ds(base, C)], pi.at[b])
        pltpu.async_copy(tileT.at[xi.at[b]], bufA.at[b], semA[b])
        pltpu.async_copy(typeT.at[ti.at[b]], bufB.at[b], semB[b])
        pltpu.async_copy(posT.at[pi.at[b]], bufC.at[b], semC[b])

    def wait_gathers(b):
        pltpu.make_async_copy(tileT.at[xi.at[b]], bufA.at[b], semA[b]).wait()
        pltpu.make_async_copy(typeT.at[ti.at[b]], bufB.at[b], semB[b]).wait()
        pltpu.make_async_copy(posT.at[pi.at[b]], bufC.at[b], semC[b]).wait()

    def wait_out(gg, b):
        base = w0 + gg * C
        pltpu.make_async_copy(bufA.at[b], out_hbm.at[pl.ds(base, C)],
                              semO[b]).wait()

    def compute(b):
        def one_token(t):
            vs = []
            for d8 in range(8):
                sl = pl.ds(d8 * 16, 16)
                vs.append(bufA[b, t, sl] + bufB[b, t, sl] + bufC[b, t, sl])
            mean = _hsum(_tree_sum(vs)) * (1.0 / 128.0)
            var = _hsum(_tree_sum([v * v for v in vs])) * (1.0 / 128.0)
            var = var - mean * mean
            rstd = _rsqrt(var + EPS)
            for d8 in range(8):
                sl = pl.ds(d8 * 16, 16)
                bufA[b, t, sl] = (vs[d8] - mean) * rstd * gv[sl] + bv[sl]

        def pair_body(i, _):
            one_token(2 * i)
            one_token(2 * i + 1)
            return 0

        lax.fori_loop(0, C // 2, pair_body, 0)

    fire(0, 0)

    def pair_body(i, _):
        for b in (0, 1):
            gg = 2 * i + b
            wait_gathers(b)
            @pl.when(gg >= 1)
            def _():
                wait_out(gg - 1, 1 - b)

            @pl.when(gg + 1 < NCHUNK)
            def _():
                fire(gg + 1, 1 - b)

            compute(b)
            base = w0 + gg * C
            pltpu.async_copy(bufA.at[b], out_hbm.at[pl.ds(base, C)], semO[b])
        return 0

    lax.fori_loop(0, NCHUNK // 2, pair_body, 0)
    wait_out(NCHUNK - 1, 1)


@jax.jit
def _run(x, tt, pp, tileT, typeT, posT, gamma, beta):
    mesh = plsc.VectorSubcoreMesh(core_axis_name="c", subcore_axis_name="s")
    kfn = pl.kernel(
        _sc_body,
        mesh=mesh,
        out_type=jax.ShapeDtypeStruct((N, D), jnp.float32),
        scratch_types=[
            pltpu.VMEM((2, C), jnp.int32),
            pltpu.VMEM((2, C), jnp.int32),
            pltpu.VMEM((2, C), jnp.int32),
            pltpu.VMEM((2, C, D), jnp.float32),
            pltpu.VMEM((2, C, D), jnp.float32),
            pltpu.VMEM((2, C, D), jnp.float32),
            pltpu.VMEM((D,), jnp.float32),
            pltpu.VMEM((D,), jnp.float32),
            pltpu.SemaphoreType.DMA,
            pltpu.SemaphoreType.DMA,
            pltpu.SemaphoreType.DMA,
            pltpu.SemaphoreType.DMA,
            pltpu.SemaphoreType.DMA,
            pltpu.SemaphoreType.DMA,
            pltpu.SemaphoreType.DMA,
            pltpu.SemaphoreType.DMA,
        ],
    )
    return kfn(x, tt, pp, tileT, typeT, posT, gamma, beta)


def kernel(x, token_type_ids, pos_ids, tile_table, type_table, pos_table,
           gamma, beta):
    xf = x.reshape(N).astype(jnp.int32)
    tf = token_type_ids.reshape(N).astype(jnp.int32)
    pf = pos_ids.reshape(N).astype(jnp.int32)
    out = _run(xf, tf, pf, tile_table, type_table, pos_table, gamma, beta)
    return out.reshape(B, L, D)
